# baseline (speedup 1.0000x reference)
import functools

import jax
import jax.numpy as jnp
from jax import lax
from jax.experimental import pallas as pl
from jax.experimental.pallas import tpu as pltpu
from jax.experimental.pallas import tpu_sc as plsc

DIM = 32
N = 10000
NG = 64
NPAD = 10016
SLOPE = (1.0 / 8.0 + 1.0 / 3.0) / 2.0
EPS = 1e-5

E1 = 160000
E1P = 163840
E3 = 10000
E3P = 16384
E3D = 20000
E3DP = 20480

NW = 32
CHUNK = 128


def _rrelu(v):
    return jnp.where(v >= 0, v, SLOPE * v)


def _gather_group_size(cw, D):
    for bc in (8, 5, 4, 3, 2, 1):
        if cw % bc == 0 and 2 * bc * CHUNK * D * 4 <= 300 * 1024:
            return bc
    return 1


def _sc_gather(table, idx2, EP, D):
    per_w = EP // NW
    cw = per_w // CHUNK
    bc = _gather_group_size(cw, D)
    ng = cw // bc
    mesh = plsc.VectorSubcoreMesh(core_axis_name="c", subcore_axis_name="s")

    @functools.partial(
        pl.kernel,
        out_type=jax.ShapeDtypeStruct((EP, D), jnp.float32),
        mesh=mesh,
        compiler_params=pltpu.CompilerParams(use_tc_tiling_on_sc=False),
        scratch_types=[
            pltpu.VMEM((2, bc, CHUNK), jnp.int32),
            pltpu.VMEM((2, bc * CHUNK, D), jnp.float32),
            pltpu.SemaphoreType.DMA((2,)),
            pltpu.SemaphoreType.DMA((2,)),
            pltpu.SemaphoreType.DMA((2,)),
        ],
    )
    def k(table_hbm, idx_hbm, out_hbm, idx_v, rows_v, si, sg, sw):
        wid = lax.axis_index("s") * 2 + lax.axis_index("c")
        base = wid * per_w
        brow = base // CHUNK

        def start_idx(g, b):
            pltpu.async_copy(idx_hbm.at[pl.ds(brow + g * bc, bc)],
                             idx_v.at[b], si.at[b])

        def fire_gathers(b):
            for j in range(bc):
                pltpu.async_copy(table_hbm.at[idx_v.at[b, j]],
                                 rows_v.at[b, pl.ds(j * CHUNK, CHUNK)], sg.at[b])

        def drain_gathers(b):
            for j in range(bc):
                pltpu.make_async_copy(
                    table_hbm.at[idx_v.at[b, j]],
                    rows_v.at[b, pl.ds(j * CHUNK, CHUNK)], sg.at[b]).wait()

        def start_write(g, b):
            pltpu.async_copy(rows_v.at[b],
                             out_hbm.at[pl.ds(base + g * bc * CHUNK, bc * CHUNK)],
                             sw.at[b])

        def wait_write(g, b):
            pltpu.make_async_copy(
                rows_v.at[b],
                out_hbm.at[pl.ds(base + g * bc * CHUNK, bc * CHUNK)],
                sw.at[b]).wait()

        start_idx(0, 0)

        def body(g, carry):
            b = lax.rem(g, 2)

            @pl.when(b == 0)
            def _():
                _one_group(g, 0)

            @pl.when(b == 1)
            def _():
                _one_group(g, 1)
            return carry

        def _one_group(g, b):
            pltpu.make_async_copy(idx_hbm.at[pl.ds(brow, bc)], idx_v.at[b],
                                  si.at[b]).wait()

            @pl.when(g + 1 < ng)
            def _():
                start_idx(g + 1, 1 - b)

            @pl.when(g >= 2)
            def _():
                wait_write(g - 2, b)
            fire_gathers(b)
            drain_gathers(b)
            start_write(g, b)

        lax.fori_loop(0, ng, body, 0)

        if ng >= 2:
            wait_write(ng - 2, (ng - 2) % 2)
        wait_write(ng - 1, (ng - 1) % 2)

    return k(table, idx2)


def _sc_scatter_add(vals, idx, EP, D, use_ones):
    per_w = EP // NW
    cw = per_w // CHUNK
    rps = NPAD // 16
    mesh = plsc.VectorSubcoreMesh(core_axis_name="c", subcore_axis_name="s")

    bc = _gather_group_size(cw, D)
    ng = cw // bc

    @functools.partial(
        pl.kernel,
        out_type=jax.ShapeDtypeStruct((2, NPAD, D), jnp.float32),
        mesh=mesh,
        compiler_params=pltpu.CompilerParams(use_tc_tiling_on_sc=False),
        scratch_types=[
            pltpu.VMEM((2, bc, CHUNK), jnp.int32),
            pltpu.VMEM((2, bc * CHUNK, D), jnp.float32),
            pltpu.VMEM((rps, D), jnp.float32),
            pltpu.VMEM_SHARED((NPAD, D), jnp.float32),
            pltpu.SemaphoreType.DMA((2,)),
            pltpu.SemaphoreType.DMA((2,)),
            pltpu.SemaphoreType.DMA((2,)),
        ],
    )
    def k(vals_hbm, idx_hbm, out_hbm, idx_v, vals_v, stripe_v, acc_sh, si, sv, sa):
        c = lax.axis_index("c")
        s = lax.axis_index("s")
        wid = s * 2 + c
        base = wid * per_w
        brow = base // CHUNK

        fill = jnp.zeros((16,), jnp.float32)

        def zrow(i, carry):
            def zcol(j, carry2):
                stripe_v[i, pl.ds(j * 16, 16)] = fill
                return carry2
            return lax.fori_loop(0, D // 16, zcol, carry)

        lax.fori_loop(0, rps, zrow, 0)
        pltpu.sync_copy(stripe_v, acc_sh.at[pl.ds(s * rps, rps)])

        if use_ones:
            one = jnp.ones((16,), jnp.float32)

            def orow(i, carry):
                def ocol(j, carry2):
                    vals_v[0, i, pl.ds(j * 16, 16)] = one
                    return carry2
                return lax.fori_loop(0, D // 16, ocol, carry)

            lax.fori_loop(0, CHUNK, orow, 0)

        plsc.subcore_barrier()

        def start_idx(g, b):
            pltpu.async_copy(idx_hbm.at[pl.ds(brow + g * bc, bc)],
                             idx_v.at[b], si.at[b])

        def wait_idx(b):
            pltpu.make_async_copy(idx_hbm.at[pl.ds(brow, bc)], idx_v.at[b],
                                  si.at[b]).wait()

        def start_vals(g, b):
            pltpu.async_copy(vals_hbm.at[pl.ds(base + g * bc * CHUNK, bc * CHUNK)],
                             vals_v.at[b], sv.at[b])

        def wait_vals(b):
            pltpu.make_async_copy(
                vals_hbm.at[pl.ds(base, bc * CHUNK)], vals_v.at[b],
                sv.at[b]).wait()

        start_idx(0, 0)
        if not use_ones:
            start_vals(0, 0)

        def _one_group(g, b):
            wait_idx(b)
            if not use_ones:
                wait_vals(b)

            @pl.when(g + 1 < ng)
            def _():
                start_idx(g + 1, 1 - b)
                if not use_ones:
                    start_vals(g + 1, 1 - b)

            descs = []
            for j in range(bc):
                vb = 0 if use_ones else b
                src = vals_v.at[vb, pl.ds(0 if use_ones else j * CHUNK, CHUNK)]
                descs.append(pltpu.async_copy(
                    src, acc_sh.at[idx_v.at[b, j]], sa.at[b], add=True))
            for dd in descs:
                dd.wait()

        def body(g, carry):
            b = lax.rem(g, 2)

            @pl.when(b == 0)
            def _():
                _one_group(g, 0)

            @pl.when(b == 1)
            def _():
                _one_group(g, 1)
            return carry

        lax.fori_loop(0, ng, body, 0)
        plsc.subcore_barrier()
        pltpu.sync_copy(acc_sh.at[pl.ds(s * rps, rps)], stripe_v)
        pltpu.sync_copy(stripe_v, out_hbm.at[c, pl.ds(s * rps, rps)])

    return k(vals, idx)


def _sc_degree(idx, EP):
    D = 16
    per_w = EP // NW
    cw = per_w // CHUNK
    rps = NPAD // 16
    mesh = plsc.VectorSubcoreMesh(core_axis_name="c", subcore_axis_name="s")
    bc = _gather_group_size(cw, D)
    ng = cw // bc

    @functools.partial(
        pl.kernel,
        out_type=jax.ShapeDtypeStruct((2, NPAD, D), jnp.float32),
        mesh=mesh,
        compiler_params=pltpu.CompilerParams(use_tc_tiling_on_sc=False),
        scratch_types=[
            pltpu.VMEM((2, bc, CHUNK), jnp.int32),
            pltpu.VMEM((CHUNK, D), jnp.float32),
            pltpu.VMEM((rps, D), jnp.float32),
            pltpu.VMEM_SHARED((NPAD, D), jnp.float32),
            pltpu.SemaphoreType.DMA((2,)),
            pltpu.SemaphoreType.DMA((2,)),
        ],
    )
    def k(idx_hbm, out_hbm, idx_v, ones_v, stripe_v, acc_sh, si, sa):
        c = lax.axis_index("c")
        s = lax.axis_index("s")
        wid = s * 2 + c
        base = wid * per_w
        brow = base // CHUNK

        fill = jnp.zeros((16,), jnp.float32)

        def zrow(i, carry):
            stripe_v[i, pl.ds(0, 16)] = fill
            return carry

        lax.fori_loop(0, rps, zrow, 0)
        pltpu.sync_copy(stripe_v, acc_sh.at[pl.ds(s * rps, rps)])

        one = jnp.ones((16,), jnp.float32)

        def orow(i, carry):
            ones_v[i, pl.ds(0, 16)] = one
            return carry

        lax.fori_loop(0, CHUNK, orow, 0)
        plsc.subcore_barrier()

        def start_idx(g, b):
            pltpu.async_copy(idx_hbm.at[pl.ds(brow + g * bc, bc)],
                             idx_v.at[b], si.at[b])

        def wait_idx(b):
            pltpu.make_async_copy(idx_hbm.at[pl.ds(brow, bc)], idx_v.at[b],
                                  si.at[b]).wait()

        start_idx(0, 0)

        def _one_group(g, b):
            wait_idx(b)

            @pl.when(g + 1 < ng)
            def _():
                start_idx(g + 1, 1 - b)

            descs = []
            for j in range(bc):
                descs.append(pltpu.async_copy(
                    ones_v, acc_sh.at[idx_v.at[b, j]], sa.at[b], add=True))
            for dd in descs:
                dd.wait()

        def body(g, carry):
            b = lax.rem(g, 2)

            @pl.when(b == 0)
            def _():
                _one_group(g, 0)

            @pl.when(b == 1)
            def _():
                _one_group(g, 1)
            return carry

        lax.fori_loop(0, ng, body, 0)
        plsc.subcore_barrier()
        pltpu.sync_copy(acc_sh.at[pl.ds(s * rps, rps)], stripe_v)
        pltpu.sync_copy(stripe_v, out_hbm.at[c, pl.ds(s * rps, rps)])

    return k(idx)


def _colstats(x, TR):
    n, d = x.shape
    grid = (n // TR,)

    def body(x_ref, o_ref):
        @pl.when(pl.program_id(0) == 0)
        def _():
            o_ref[...] = jnp.zeros_like(o_ref)
        xv = x_ref[...]
        s1 = jnp.sum(xv, axis=0, keepdims=True)
        s2 = jnp.sum(xv * xv, axis=0, keepdims=True)
        o_ref[0:1, :] += s1
        o_ref[1:2, :] += s2

    return pl.pallas_call(
        body,
        grid=grid,
        in_specs=[pl.BlockSpec((TR, d), lambda i: (i, 0))],
        out_specs=pl.BlockSpec((8, d), lambda i: (0, 0)),
        out_shape=jax.ShapeDtypeStruct((8, d), jnp.float32),
    )(x)


def _bn_from_stats(stats_ref, nrows):
    s1 = stats_ref[0:1, :]
    s2 = stats_ref[1:2, :]
    mean = s1 / nrows
    var = s2 / nrows - mean * mean
    inv = lax.rsqrt(var + EPS)
    return mean, inv


def _pre_node(x, stats, g, b, WT, lb, TR):
    n, d = x.shape
    do = WT.shape[1]

    def body(x_ref, st_ref, g_ref, b_ref, w_ref, lb_ref, o_ref):
        mean, inv = _bn_from_stats(st_ref, float(n))
        xn = (x_ref[...] - mean) * inv * g_ref[...] + b_ref[...]
        y = _bdot(xn, w_ref[...]) + lb_ref[...]
        o_ref[...] = _rrelu(y)

    return pl.pallas_call(
        body,
        grid=(n // TR,),
        in_specs=[
            pl.BlockSpec((TR, d), lambda i: (i, 0)),
            pl.BlockSpec((8, d), lambda i: (0, 0)),
            pl.BlockSpec((1, d), lambda i: (0, 0)),
            pl.BlockSpec((1, d), lambda i: (0, 0)),
            pl.BlockSpec((d, do), lambda i: (0, 0)),
            pl.BlockSpec((1, do), lambda i: (0, 0)),
        ],
        out_specs=pl.BlockSpec((TR, do), lambda i: (i, 0)),
        out_shape=jax.ShapeDtypeStruct((n, do), jnp.float32),
    )(x, stats, g, b, WT, lb)


def _lin_rrelu(x, WT, lb, TR):
    n, d = x.shape
    do = WT.shape[1]

    def body(x_ref, w_ref, lb_ref, o_ref):
        y = _bdot(x_ref[...], w_ref[...])
        o_ref[...] = _rrelu(y + lb_ref[...])

    return pl.pallas_call(
        body,
        grid=(n // TR,),
        in_specs=[
            pl.BlockSpec((TR, d), lambda i: (i, 0)),
            pl.BlockSpec((d, do), lambda i: (0, 0)),
            pl.BlockSpec((1, do), lambda i: (0, 0)),
        ],
        out_specs=pl.BlockSpec((TR, do), lambda i: (i, 0)),
        out_shape=jax.ShapeDtypeStruct((n, do), jnp.float32),
    )(x, WT, lb)


def _bf(v):
    return v.astype(jnp.bfloat16)


def _bdot(a, b):
    return jnp.dot(_bf(a), _bf(b), preferred_element_type=jnp.float32)


def _msg(G, ea, nnW, K, D, TR, lin=None):
    ep = G.shape[0]
    IG = 128 // D
    J = (D * D) // 128
    Wg = _bf(nnW.reshape(J, 128, K).transpose(0, 2, 1))
    Sg = _bf((lax.broadcasted_iota(jnp.int32, (IG, 128), 1) // D
              == lax.broadcasted_iota(jnp.int32, (IG, 128), 0)).astype(jnp.float32))

    ne = ea.shape[0]
    KE = ea.shape[1]
    has_lin = lin is not None

    def body(g_ref, ea_ref, lw_ref, lb_ref, wg_ref, sg_ref, o_ref):
        if has_lin:
            eav = _rrelu(_bdot(ea_ref[...], lw_ref[...]) + lb_ref[...])
        else:
            eav = ea_ref[...]
        eab = _bf(eav)
        gb = _bf(g_ref[...])
        acc = jnp.zeros((TR, 128), jnp.float32)
        for j in range(J):
            p = _bf(jnp.dot(eab, wg_ref[j], preferred_element_type=jnp.float32)
                    ).astype(jnp.float32)
            m = jnp.dot(gb[:, j * IG:(j + 1) * IG], sg_ref[...],
                        preferred_element_type=jnp.float32)
            acc = acc + p * m
        sh = 64
        while sh >= D:
            acc = acc + jnp.roll(acc, -sh, axis=1)
            sh //= 2
        o_ref[...] = acc[:, :D]

    ng = -(-ne // TR)
    lw = lin[0] if has_lin else jnp.zeros((KE, K), jnp.float32)
    lb = lin[1] if has_lin else jnp.zeros((1, K), jnp.float32)
    return pl.pallas_call(
        body,
        grid=(ng,),
        in_specs=[
            pl.BlockSpec((TR, D), lambda i: (i, 0)),
            pl.BlockSpec((TR, KE), lambda i: (i, 0)),
            pl.BlockSpec((KE, K), lambda i: (0, 0)),
            pl.BlockSpec((1, K), lambda i: (0, 0)),
            pl.BlockSpec((J, K, 128), lambda i: (0, 0, 0)),
            pl.BlockSpec((IG, 128), lambda i: (0, 0)),
        ],
        out_specs=pl.BlockSpec((TR, D), lambda i: (i, 0)),
        out_shape=jax.ShapeDtypeStruct((ep, D), jnp.float32),
    )(G, ea, lw, lb, Wg, Sg)


def _agg_gru(s0, s1, d0, d1, cb, h, WihT, WhhT, bih, bhh, D, TR):
    n = h.shape[0]

    def body(s0_ref, s1_ref, d0_ref, d1_ref, cb_ref, h_ref,
             wi_ref, wh_ref, bi_ref, bh_ref, o_ref):
        deg = jnp.maximum(d0_ref[:, 0:1] + d1_ref[:, 0:1], 1.0)
        m = _rrelu((s0_ref[...] + s1_ref[...]) / deg + cb_ref[...])
        hv = h_ref[...]
        gi = _bdot(m, wi_ref[...]) + bi_ref[...]
        gh = _bdot(hv, wh_ref[...]) + bh_ref[...]
        r = jax.nn.sigmoid(gi[:, :D] + gh[:, :D])
        z = jax.nn.sigmoid(gi[:, D:2 * D] + gh[:, D:2 * D])
        nn = jnp.tanh(gi[:, 2 * D:] + r * gh[:, 2 * D:])
        o_ref[...] = (1.0 - z) * nn + z * hv

    return pl.pallas_call(
        body,
        grid=(n // TR,),
        in_specs=[
            pl.BlockSpec((TR, D), lambda i: (i, 0)),
            pl.BlockSpec((TR, D), lambda i: (i, 0)),
            pl.BlockSpec((TR, 16), lambda i: (i, 0)),
            pl.BlockSpec((TR, 16), lambda i: (i, 0)),
            pl.BlockSpec((1, D), lambda i: (0, 0)),
            pl.BlockSpec((TR, D), lambda i: (i, 0)),
            pl.BlockSpec((D, 3 * D), lambda i: (0, 0)),
            pl.BlockSpec((D, 3 * D), lambda i: (0, 0)),
            pl.BlockSpec((1, 3 * D), lambda i: (0, 0)),
            pl.BlockSpec((1, 3 * D), lambda i: (0, 0)),
        ],
        out_specs=pl.BlockSpec((TR, D), lambda i: (i, 0)),
        out_shape=jax.ShapeDtypeStruct((n, D), jnp.float32),
    )(s0, s1, d0, d1, cb, h, WihT, WhhT, bih, bhh)


def _mid(h, stats, g, b, W1T, b1, W2T, b2, TR):
    n, d = h.shape
    do = W1T.shape[1]

    def body(h_ref, st_ref, g_ref, b_ref, w1_ref, b1_ref, w2_ref, b2_ref, o_ref):
        mean, inv = _bn_from_stats(st_ref, float(n))
        xn = (h_ref[...] - mean) * inv * g_ref[...] + b_ref[...]
        y = _rrelu(_bdot(xn, w1_ref[...]) + b1_ref[...])
        o_ref[...] = _rrelu(_bdot(y, w2_ref[...]) + b2_ref[...])

    return pl.pallas_call(
        body,
        grid=(n // TR,),
        in_specs=[
            pl.BlockSpec((TR, d), lambda i: (i, 0)),
            pl.BlockSpec((8, d), lambda i: (0, 0)),
            pl.BlockSpec((1, d), lambda i: (0, 0)),
            pl.BlockSpec((1, d), lambda i: (0, 0)),
            pl.BlockSpec((d, do), lambda i: (0, 0)),
            pl.BlockSpec((1, do), lambda i: (0, 0)),
            pl.BlockSpec((do, do), lambda i: (0, 0)),
            pl.BlockSpec((1, do), lambda i: (0, 0)),
        ],
        out_specs=pl.BlockSpec((TR, do), lambda i: (i, 0)),
        out_shape=jax.ShapeDtypeStruct((n, do), jnp.float32),
    )(h, stats, g, b, W1T, b1, W2T, b2)


def _set2set(out, batch2d, WihT, WhhT, bih, bhh, TR):
    n, d = out.shape

    def body(o_ref, b_ref, wi_ref, wh_ref, bi_ref, bh_ref, qs_out,
             hx_s, cx_s, qs_s, emax_s, den_s, r_s, e_s):
        s = pl.program_id(0)
        p = pl.program_id(1)
        t = pl.program_id(2)
        nt = pl.num_programs(2)

        @pl.when((s == 0) & (p == 0) & (t == 0))
        def _():
            hx_s[...] = jnp.zeros_like(hx_s)
            cx_s[...] = jnp.zeros_like(cx_s)
            qs_s[...] = jnp.zeros_like(qs_s)

        @pl.when((p == 0) & (t == 0))
        def _():
            g = (_bdot(qs_s[...], wi_ref[...]) + bi_ref[...]
                 + _bdot(hx_s[...], wh_ref[...]) + bh_ref[...])
            i = jax.nn.sigmoid(g[:, :d])
            f = jax.nn.sigmoid(g[:, d:2 * d])
            gg = jnp.tanh(g[:, 2 * d:3 * d])
            o = jax.nn.sigmoid(g[:, 3 * d:])
            c2 = f * cx_s[...] + i * gg
            cx_s[...] = c2
            hx_s[...] = o * jnp.tanh(c2)
            emax_s[...] = jnp.full_like(emax_s, -1e30)
            den_s[...] = jnp.zeros_like(den_s)
            r_s[...] = jnp.zeros_like(r_s)

        ov = o_ref[...]
        oh = (b_ref[...] == lax.broadcasted_iota(jnp.int32, (TR, NG), 1)
              ).astype(jnp.float32)

        @pl.when(p == 0)
        def _():
            qb = jnp.dot(oh, hx_s[...], preferred_element_type=jnp.float32, precision=lax.Precision.HIGHEST)
            e0 = jnp.sum(ov * qb, axis=1, keepdims=True)
            e_s[pl.ds(t * TR, TR), :] = e0
            em = jnp.max(jnp.where(oh > 0, e0, -1e30), axis=0, keepdims=True)
            emax_s[0:1, :] = jnp.maximum(emax_s[0:1, :], em)

        e = e_s[pl.ds(t * TR, TR), :]

        @pl.when(p == 1)
        def _():
            emb = jnp.dot(oh, emax_s[0:1, :].reshape(NG, 1),
                          preferred_element_type=jnp.float32, precision=lax.Precision.HIGHEST)
            ex = jnp.exp(e - emb)
            dpart = lax.dot_general(ex, oh, (((0,), (0,)), ((), ())),
                                    preferred_element_type=jnp.float32, precision=lax.Precision.HIGHEST)
            den_s[0:1, :] += dpart

        @pl.when(p == 2)
        def _():
            emb = jnp.dot(oh, emax_s[0:1, :].reshape(NG, 1),
                          preferred_element_type=jnp.float32, precision=lax.Precision.HIGHEST)
            ex = jnp.exp(e - emb)
            dnb = jnp.dot(oh, den_s[0:1, :].reshape(NG, 1),
                          preferred_element_type=jnp.float32, precision=lax.Precision.HIGHEST)
            a = ex / (dnb + 1e-16)
            rpart = lax.dot_general(oh, a * ov, (((0,), (0,)), ((), ())),
                                    preferred_element_type=jnp.float32, precision=lax.Precision.HIGHEST)
            r_s[...] += rpart

            @pl.when(t == nt - 1)
            def _():
                qs_s[:, :d] = hx_s[...]
                qs_s[:, d:] = r_s[...]
                qs_out[...] = qs_s[...]

    return pl.pallas_call(
        body,
        grid=(3, 3, n // TR),
        in_specs=[
            pl.BlockSpec((TR, d), lambda s, p, t: (t, 0)),
            pl.BlockSpec((TR, 1), lambda s, p, t: (t, 0)),
            pl.BlockSpec((2 * d, 4 * d), lambda s, p, t: (0, 0)),
            pl.BlockSpec((d, 4 * d), lambda s, p, t: (0, 0)),
            pl.BlockSpec((1, 4 * d), lambda s, p, t: (0, 0)),
            pl.BlockSpec((1, 4 * d), lambda s, p, t: (0, 0)),
        ],
        out_specs=pl.BlockSpec((NG, 2 * d), lambda s, p, t: (0, 0)),
        out_shape=jax.ShapeDtypeStruct((NG, 2 * d), jnp.float32),
        scratch_shapes=[
            pltpu.VMEM((NG, d), jnp.float32),
            pltpu.VMEM((NG, d), jnp.float32),
            pltpu.VMEM((NG, 2 * d), jnp.float32),
            pltpu.VMEM((8, NG), jnp.float32),
            pltpu.VMEM((8, NG), jnp.float32),
            pltpu.VMEM((NG, d), jnp.float32),
            pltpu.VMEM((n, 1), jnp.float32),
        ],
    )(out, batch2d, WihT, WhhT, bih, bhh)


def _expand_cat(out, batch2d, qs, TR):
    n, d = out.shape

    def body(o_ref, b_ref, qs_ref, cat_ref):
        oh = (b_ref[...] == lax.broadcasted_iota(jnp.int32, (TR, NG), 1)
              ).astype(jnp.float32)
        pooln = jnp.dot(oh, qs_ref[...], preferred_element_type=jnp.float32, precision=lax.Precision.HIGHEST)
        cat_ref[:, :d] = o_ref[...]
        cat_ref[:, d:] = pooln

    return pl.pallas_call(
        body,
        grid=(n // TR,),
        in_specs=[
            pl.BlockSpec((TR, d), lambda i: (i, 0)),
            pl.BlockSpec((TR, 1), lambda i: (i, 0)),
            pl.BlockSpec((NG, 2 * d), lambda i: (0, 0)),
        ],
        out_specs=pl.BlockSpec((TR, 3 * d), lambda i: (i, 0)),
        out_shape=jax.ShapeDtypeStruct((n, 3 * d), jnp.float32),
    )(out, batch2d, qs)


def _yhat_tile(t0cat, t1):
    d = 64
    t0 = t0cat[:, :d]
    p3 = t0cat[:, d:]
    return jnp.concatenate(
        [(t0 + t1) * 0.5, t0 * t1, (t0 - t1) ** 2, p3], axis=1)


def _final_stats(T0cat, T1, TR):
    n = T0cat.shape[0]

    def body(t0_ref, t1_ref, o_ref):
        @pl.when(pl.program_id(0) == 0)
        def _():
            o_ref[...] = jnp.zeros_like(o_ref)
        y = _yhat_tile(t0_ref[...], t1_ref[...])
        o_ref[0:1, :] += jnp.sum(y, axis=0, keepdims=True)
        o_ref[1:2, :] += jnp.sum(y * y, axis=0, keepdims=True)

    return pl.pallas_call(
        body,
        grid=(n // TR,),
        in_specs=[
            pl.BlockSpec((TR, 192), lambda i: (i, 0)),
            pl.BlockSpec((TR, 64), lambda i: (i, 0)),
        ],
        out_specs=pl.BlockSpec((8, 320), lambda i: (0, 0)),
        out_shape=jax.ShapeDtypeStruct((8, 320), jnp.float32),
    )(T0cat, T1)


def _final_head(T0cat, T1, stats, ng, nb, ea3, WwT, WbT, TR):
    n = T0cat.shape[0]

    def body(t0_ref, t1_ref, st_ref, g_ref, b_ref, ea_ref, ww_ref, wb_ref, o_ref):
        y = _yhat_tile(t0_ref[...], t1_ref[...])
        mean, inv = _bn_from_stats(st_ref, float(n))
        y = (y - mean) * inv * g_ref[...] + b_ref[...]
        eav = ea_ref[...]
        w = _bdot(eav, ww_ref[...])
        bb = _bdot(eav, wb_ref[...])
        o_ref[...] = jnp.sum(y * w, axis=1, keepdims=True) + bb

    return pl.pallas_call(
        body,
        grid=(n // TR,),
        in_specs=[
            pl.BlockSpec((TR, 192), lambda i: (i, 0)),
            pl.BlockSpec((TR, 64), lambda i: (i, 0)),
            pl.BlockSpec((8, 320), lambda i: (0, 0)),
            pl.BlockSpec((1, 320), lambda i: (0, 0)),
            pl.BlockSpec((1, 320), lambda i: (0, 0)),
            pl.BlockSpec((TR, 8), lambda i: (i, 0)),
            pl.BlockSpec((8, 320), lambda i: (0, 0)),
            pl.BlockSpec((8, 1), lambda i: (0, 0)),
        ],
        out_specs=pl.BlockSpec((TR, 1), lambda i: (i, 0)),
        out_shape=jax.ShapeDtypeStruct((n, 1), jnp.float32),
    )(T0cat, T1, stats, ng, nb, ea3, WwT, WbT)


def _pad_idx(a, ep, fill):
    return jnp.pad(a, (0, ep - a.shape[0]), constant_values=fill)


def kernel(x, edge_attr, edge_attr3, params, edge_index, edge_index3, batch):
    p = params
    f32 = jnp.float32

    src = _pad_idx(edge_index[0].astype(jnp.int32), E1P, 0).reshape(-1, CHUNK)
    dst = _pad_idx(edge_index[1].astype(jnp.int32), E1P, N).reshape(-1, CHUNK)
    src3 = edge_index3[0].astype(jnp.int32)
    dst3 = edge_index3[1].astype(jnp.int32)
    src3d = _pad_idx(jnp.concatenate([src3, dst3]), E3DP, 0).reshape(-1, CHUNK)
    dst3d = _pad_idx(jnp.concatenate([dst3, src3]), E3DP, N).reshape(-1, CHUNK)
    src3p = _pad_idx(src3, E3P, 0).reshape(-1, CHUNK)
    dst3p = _pad_idx(dst3, E3P, 0).reshape(-1, CHUNK)

    Wt1 = p['nn1_W']
    Wt2 = p['nn2_W']

    def row(v):
        return v.reshape(1, -1).astype(f32)

    ea3d = jnp.concatenate([edge_attr3, edge_attr3], axis=0)

    xst = _colstats(x, 1000)
    out = _pre_node(x, xst, row(p['norm_x_g']), row(p['norm_x_b']),
                    p['lin_node_W'].T, row(p['lin_node_b']), 1000)

    degp = _sc_degree(dst, E1P)
    d0 = degp[0, :N]
    d1 = degp[1, :N]
    h = out
    for _ in range(2):
        G = _sc_gather(out, src, E1P, DIM)
        msg = _msg(G, edge_attr, Wt1, 12, DIM, 512,
                   lin=(p['lin_edge_W'].T, row(p['lin_edge_b'])))
        sp = _sc_scatter_add(msg, dst, E1P, DIM, False)
        h = _agg_gru(sp[0, :N], sp[1, :N], d0, d1, row(p['conv1_b']), h,
                     p['gru1_Wih'].T, p['gru1_Whh'].T,
                     row(p['gru1_bih']), row(p['gru1_bhh']), DIM, 1000)
        out = h

    hst = _colstats(out, 1000)
    out = _mid(out, hst, row(p['cov_bn_g']), row(p['cov_bn_b']),
               p['cov_l1_W'].T, row(p['cov_l1_b']),
               p['cov_l2_W'].T, row(p['cov_l2_b']), 1000)

    degp3 = _sc_degree(dst3d, E3DP)
    d30 = degp3[0, :N]
    d31 = degp3[1, :N]
    h = out
    for _ in range(2):
        G = _sc_gather(out, src3d, E3DP, 2 * DIM)
        msg = _msg(G, ea3d, Wt2, 8, 2 * DIM, 512)
        sp = _sc_scatter_add(msg, dst3d, E3DP, 2 * DIM, False)
        h = _agg_gru(sp[0, :N], sp[1, :N], d30, d31, row(p['conv2_b']), h,
                     p['gru2_Wih'].T, p['gru2_Whh'].T,
                     row(p['gru2_bih']), row(p['gru2_bhh']), 2 * DIM, 1000)
        out = h

    batch2d = batch.astype(jnp.int32).reshape(N, 1)
    qs = _set2set(out, batch2d, p['s2s_Wih'].T, p['s2s_Whh'].T,
                  row(p['s2s_bih']), row(p['s2s_bhh']), 2000)
    cat = _expand_cat(out, batch2d, qs, 1000)

    T0cat = _sc_gather(cat, src3p, E3P, 192)[:N]
    T1 = _sc_gather(out, dst3p, E3P, 2 * DIM)[:N]

    fst = _final_stats(T0cat, T1, 1000)
    y = _final_head(T0cat, T1, fst, row(p['norm_g']), row(p['norm_b']),
                    edge_attr3, p['lin_weight_W'].T, p['lin_bias_W'].T, 1000)
    return y[:, 0]


# speedup vs baseline: 1.1236x; 1.1236x over previous
import functools

import jax
import jax.numpy as jnp
from jax import lax
from jax.experimental import pallas as pl
from jax.experimental.pallas import tpu as pltpu
from jax.experimental.pallas import tpu_sc as plsc

DIM = 32
N = 10000
NG = 64
NPAD = 10016
SLOPE = (1.0 / 8.0 + 1.0 / 3.0) / 2.0
EPS = 1e-5

E1 = 160000
E1P = 163840
E3 = 10000
E3P = 12288
E3D = 20000
E3DP = 20480

NW = 32
CHUNK = 128


def _rrelu(v):
    return jnp.where(v >= 0, v, SLOPE * v)


def _gather_group_size(cw, D):
    for bc in (8, 5, 4, 3, 2, 1):
        if cw % bc == 0 and 2 * bc * CHUNK * D * 4 <= 300 * 1024:
            return bc
    return 1


def _sc_gather(table, idx2, EP, D):
    per_w = EP // NW
    cw = per_w // CHUNK
    bc = _gather_group_size(cw, D)
    ng = cw // bc
    mesh = plsc.VectorSubcoreMesh(core_axis_name="c", subcore_axis_name="s")

    @functools.partial(
        pl.kernel,
        out_type=jax.ShapeDtypeStruct((EP, D), jnp.float32),
        mesh=mesh,
        compiler_params=pltpu.CompilerParams(use_tc_tiling_on_sc=False),
        scratch_types=[
            pltpu.VMEM((2, bc, CHUNK), jnp.int32),
            pltpu.VMEM((2, bc * CHUNK, D), jnp.float32),
            pltpu.SemaphoreType.DMA((2,)),
            pltpu.SemaphoreType.DMA((2,)),
            pltpu.SemaphoreType.DMA((2,)),
        ],
    )
    def k(table_hbm, idx_hbm, out_hbm, idx_v, rows_v, si, sg, sw):
        wid = lax.axis_index("s") * 2 + lax.axis_index("c")
        base = wid * per_w
        brow = base // CHUNK

        def start_idx(g, b):
            pltpu.async_copy(idx_hbm.at[pl.ds(brow + g * bc, bc)],
                             idx_v.at[b], si.at[b])

        def fire_gathers(b):
            for j in range(bc):
                pltpu.async_copy(table_hbm.at[idx_v.at[b, j]],
                                 rows_v.at[b, pl.ds(j * CHUNK, CHUNK)], sg.at[b])

        def drain_gathers(b):
            for j in range(bc):
                pltpu.make_async_copy(
                    table_hbm.at[idx_v.at[b, j]],
                    rows_v.at[b, pl.ds(j * CHUNK, CHUNK)], sg.at[b]).wait()

        def start_write(g, b):
            pltpu.async_copy(rows_v.at[b],
                             out_hbm.at[pl.ds(base + g * bc * CHUNK, bc * CHUNK)],
                             sw.at[b])

        def wait_write(g, b):
            pltpu.make_async_copy(
                rows_v.at[b],
                out_hbm.at[pl.ds(base + g * bc * CHUNK, bc * CHUNK)],
                sw.at[b]).wait()

        start_idx(0, 0)

        def body(g, carry):
            b = lax.rem(g, 2)

            @pl.when(b == 0)
            def _():
                _one_group(g, 0)

            @pl.when(b == 1)
            def _():
                _one_group(g, 1)
            return carry

        def _one_group(g, b):
            pltpu.make_async_copy(idx_hbm.at[pl.ds(brow, bc)], idx_v.at[b],
                                  si.at[b]).wait()

            @pl.when(g + 1 < ng)
            def _():
                start_idx(g + 1, 1 - b)

            @pl.when(g >= 2)
            def _():
                wait_write(g - 2, b)
            fire_gathers(b)
            drain_gathers(b)
            start_write(g, b)

        lax.fori_loop(0, ng, body, 0)

        if ng >= 2:
            wait_write(ng - 2, (ng - 2) % 2)
        wait_write(ng - 1, (ng - 1) % 2)

    return k(table, idx2)


def _sc_scatter_add(vals, idx, EP, D, use_ones):
    per_w = EP // NW
    cw = per_w // CHUNK
    rps = NPAD // 16
    mesh = plsc.VectorSubcoreMesh(core_axis_name="c", subcore_axis_name="s")

    bc = _gather_group_size(cw, D)
    ng = cw // bc

    @functools.partial(
        pl.kernel,
        out_type=jax.ShapeDtypeStruct((2, NPAD, D), jnp.float32),
        mesh=mesh,
        compiler_params=pltpu.CompilerParams(use_tc_tiling_on_sc=False),
        scratch_types=[
            pltpu.VMEM((2, bc, CHUNK), jnp.int32),
            pltpu.VMEM((2, bc * CHUNK, D), jnp.float32),
            pltpu.VMEM((rps, D), jnp.float32),
            pltpu.VMEM_SHARED((NPAD, D), jnp.float32),
            pltpu.SemaphoreType.DMA((2,)),
            pltpu.SemaphoreType.DMA((2,)),
            pltpu.SemaphoreType.DMA((2,)),
        ],
    )
    def k(vals_hbm, idx_hbm, out_hbm, idx_v, vals_v, stripe_v, acc_sh, si, sv, sa):
        c = lax.axis_index("c")
        s = lax.axis_index("s")
        wid = s * 2 + c
        base = wid * per_w
        brow = base // CHUNK

        fill = jnp.zeros((16,), jnp.float32)

        def zrow(i, carry):
            def zcol(j, carry2):
                stripe_v[i, pl.ds(j * 16, 16)] = fill
                return carry2
            return lax.fori_loop(0, D // 16, zcol, carry)

        lax.fori_loop(0, rps, zrow, 0)
        pltpu.sync_copy(stripe_v, acc_sh.at[pl.ds(s * rps, rps)])

        if use_ones:
            one = jnp.ones((16,), jnp.float32)

            def orow(i, carry):
                def ocol(j, carry2):
                    vals_v[0, i, pl.ds(j * 16, 16)] = one
                    return carry2
                return lax.fori_loop(0, D // 16, ocol, carry)

            lax.fori_loop(0, CHUNK, orow, 0)

        plsc.subcore_barrier()

        def start_idx(g, b):
            pltpu.async_copy(idx_hbm.at[pl.ds(brow + g * bc, bc)],
                             idx_v.at[b], si.at[b])

        def wait_idx(b):
            pltpu.make_async_copy(idx_hbm.at[pl.ds(brow, bc)], idx_v.at[b],
                                  si.at[b]).wait()

        def start_vals(g, b):
            pltpu.async_copy(vals_hbm.at[pl.ds(base + g * bc * CHUNK, bc * CHUNK)],
                             vals_v.at[b], sv.at[b])

        def wait_vals(b):
            pltpu.make_async_copy(
                vals_hbm.at[pl.ds(base, bc * CHUNK)], vals_v.at[b],
                sv.at[b]).wait()

        start_idx(0, 0)
        if not use_ones:
            start_vals(0, 0)

        def _one_group(g, b):
            wait_idx(b)
            if not use_ones:
                wait_vals(b)

            @pl.when(g + 1 < ng)
            def _():
                start_idx(g + 1, 1 - b)
                if not use_ones:
                    start_vals(g + 1, 1 - b)

            descs = []
            for j in range(bc):
                vb = 0 if use_ones else b
                src = vals_v.at[vb, pl.ds(0 if use_ones else j * CHUNK, CHUNK)]
                descs.append(pltpu.async_copy(
                    src, acc_sh.at[idx_v.at[b, j]], sa.at[b], add=True))
            for dd in descs:
                dd.wait()

        def body(g, carry):
            b = lax.rem(g, 2)

            @pl.when(b == 0)
            def _():
                _one_group(g, 0)

            @pl.when(b == 1)
            def _():
                _one_group(g, 1)
            return carry

        lax.fori_loop(0, ng, body, 0)
        plsc.subcore_barrier()
        pltpu.sync_copy(acc_sh.at[pl.ds(s * rps, rps)], stripe_v)
        pltpu.sync_copy(stripe_v, out_hbm.at[c, pl.ds(s * rps, rps)])

    return k(vals, idx)


def _sc_degree(idx, EP):
    D = 16
    per_w = EP // NW
    cw = per_w // CHUNK
    rps = NPAD // 16
    mesh = plsc.VectorSubcoreMesh(core_axis_name="c", subcore_axis_name="s")
    bc = _gather_group_size(cw, D)
    ng = cw // bc

    @functools.partial(
        pl.kernel,
        out_type=jax.ShapeDtypeStruct((2, NPAD, D), jnp.float32),
        mesh=mesh,
        compiler_params=pltpu.CompilerParams(use_tc_tiling_on_sc=False),
        scratch_types=[
            pltpu.VMEM((2, bc, CHUNK), jnp.int32),
            pltpu.VMEM((CHUNK, D), jnp.float32),
            pltpu.VMEM((rps, D), jnp.float32),
            pltpu.VMEM_SHARED((NPAD, D), jnp.float32),
            pltpu.SemaphoreType.DMA((2,)),
            pltpu.SemaphoreType.DMA((2,)),
        ],
    )
    def k(idx_hbm, out_hbm, idx_v, ones_v, stripe_v, acc_sh, si, sa):
        c = lax.axis_index("c")
        s = lax.axis_index("s")
        wid = s * 2 + c
        base = wid * per_w
        brow = base // CHUNK

        fill = jnp.zeros((16,), jnp.float32)

        def zrow(i, carry):
            stripe_v[i, pl.ds(0, 16)] = fill
            return carry

        lax.fori_loop(0, rps, zrow, 0)
        pltpu.sync_copy(stripe_v, acc_sh.at[pl.ds(s * rps, rps)])

        one = jnp.ones((16,), jnp.float32)

        def orow(i, carry):
            ones_v[i, pl.ds(0, 16)] = one
            return carry

        lax.fori_loop(0, CHUNK, orow, 0)
        plsc.subcore_barrier()

        def start_idx(g, b):
            pltpu.async_copy(idx_hbm.at[pl.ds(brow + g * bc, bc)],
                             idx_v.at[b], si.at[b])

        def wait_idx(b):
            pltpu.make_async_copy(idx_hbm.at[pl.ds(brow, bc)], idx_v.at[b],
                                  si.at[b]).wait()

        start_idx(0, 0)

        def _one_group(g, b):
            wait_idx(b)

            @pl.when(g + 1 < ng)
            def _():
                start_idx(g + 1, 1 - b)

            descs = []
            for j in range(bc):
                descs.append(pltpu.async_copy(
                    ones_v, acc_sh.at[idx_v.at[b, j]], sa.at[b], add=True))
            for dd in descs:
                dd.wait()

        def body(g, carry):
            b = lax.rem(g, 2)

            @pl.when(b == 0)
            def _():
                _one_group(g, 0)

            @pl.when(b == 1)
            def _():
                _one_group(g, 1)
            return carry

        lax.fori_loop(0, ng, body, 0)
        plsc.subcore_barrier()
        pltpu.sync_copy(acc_sh.at[pl.ds(s * rps, rps)], stripe_v)
        pltpu.sync_copy(stripe_v, out_hbm.at[c, pl.ds(s * rps, rps)])

    return k(idx)


def _colstats(x, TR):
    n, d = x.shape
    grid = (n // TR,)

    def body(x_ref, o_ref):
        @pl.when(pl.program_id(0) == 0)
        def _():
            o_ref[...] = jnp.zeros_like(o_ref)
        xv = x_ref[...]
        s1 = jnp.sum(xv, axis=0, keepdims=True)
        s2 = jnp.sum(xv * xv, axis=0, keepdims=True)
        o_ref[0:1, :] += s1
        o_ref[1:2, :] += s2

    return pl.pallas_call(
        body,
        grid=grid,
        in_specs=[pl.BlockSpec((TR, d), lambda i: (i, 0))],
        out_specs=pl.BlockSpec((8, d), lambda i: (0, 0)),
        out_shape=jax.ShapeDtypeStruct((8, d), jnp.float32),
    )(x)


def _bn_from_stats(stats_ref, nrows):
    s1 = stats_ref[0:1, :]
    s2 = stats_ref[1:2, :]
    mean = s1 / nrows
    var = s2 / nrows - mean * mean
    inv = lax.rsqrt(var + EPS)
    return mean, inv


def _pre_node(x, stats, g, b, WT, lb, TR):
    n, d = x.shape
    do = WT.shape[1]

    def body(x_ref, st_ref, g_ref, b_ref, w_ref, lb_ref, o_ref):
        mean, inv = _bn_from_stats(st_ref, float(n))
        xn = (x_ref[...] - mean) * inv * g_ref[...] + b_ref[...]
        y = _bdot(xn, w_ref[...]) + lb_ref[...]
        o_ref[...] = _rrelu(y)

    return pl.pallas_call(
        body,
        grid=(n // TR,),
        in_specs=[
            pl.BlockSpec((TR, d), lambda i: (i, 0)),
            pl.BlockSpec((8, d), lambda i: (0, 0)),
            pl.BlockSpec((1, d), lambda i: (0, 0)),
            pl.BlockSpec((1, d), lambda i: (0, 0)),
            pl.BlockSpec((d, do), lambda i: (0, 0)),
            pl.BlockSpec((1, do), lambda i: (0, 0)),
        ],
        out_specs=pl.BlockSpec((TR, do), lambda i: (i, 0)),
        out_shape=jax.ShapeDtypeStruct((n, do), jnp.float32),
    )(x, stats, g, b, WT, lb)


def _lin_rrelu(x, WT, lb, TR):
    n, d = x.shape
    do = WT.shape[1]

    def body(x_ref, w_ref, lb_ref, o_ref):
        y = _bdot(x_ref[...], w_ref[...])
        o_ref[...] = _rrelu(y + lb_ref[...])

    return pl.pallas_call(
        body,
        grid=(n // TR,),
        in_specs=[
            pl.BlockSpec((TR, d), lambda i: (i, 0)),
            pl.BlockSpec((d, do), lambda i: (0, 0)),
            pl.BlockSpec((1, do), lambda i: (0, 0)),
        ],
        out_specs=pl.BlockSpec((TR, do), lambda i: (i, 0)),
        out_shape=jax.ShapeDtypeStruct((n, do), jnp.float32),
    )(x, WT, lb)


def _bf(v):
    return v.astype(jnp.bfloat16)


def _bdot(a, b):
    return jnp.dot(_bf(a), _bf(b), preferred_element_type=jnp.float32)


def _msg(G, ea, nnW, K, D, TR, lin=None):
    ep = G.shape[0]
    IG = 128 // D
    J = (D * D) // 128
    Wg = _bf(nnW.reshape(J, 128, K).transpose(0, 2, 1))
    Sg = _bf((lax.broadcasted_iota(jnp.int32, (IG, 128), 1) // D
              == lax.broadcasted_iota(jnp.int32, (IG, 128), 0)).astype(jnp.float32))

    ne = ea.shape[0]
    KE = ea.shape[1]
    has_lin = lin is not None

    def body(g_ref, ea_ref, lw_ref, lb_ref, wg_ref, sg_ref, o_ref):
        if has_lin:
            eav = _rrelu(_bdot(ea_ref[...], lw_ref[...]) + lb_ref[...])
        else:
            eav = ea_ref[...]
        eab = _bf(eav)
        gb = _bf(g_ref[...])
        acc = jnp.zeros((TR, 128), jnp.float32)
        for j in range(J):
            p = _bf(jnp.dot(eab, wg_ref[j], preferred_element_type=jnp.float32)
                    ).astype(jnp.float32)
            m = jnp.dot(gb[:, j * IG:(j + 1) * IG], sg_ref[...],
                        preferred_element_type=jnp.float32)
            acc = acc + p * m
        sh = 64
        while sh >= D:
            acc = acc + jnp.roll(acc, -sh, axis=1)
            sh //= 2
        o_ref[...] = acc[:, :D]

    ng = -(-ne // TR)
    lw = lin[0] if has_lin else jnp.zeros((KE, K), jnp.float32)
    lb = lin[1] if has_lin else jnp.zeros((1, K), jnp.float32)
    return pl.pallas_call(
        body,
        grid=(ng,),
        in_specs=[
            pl.BlockSpec((TR, D), lambda i: (i, 0)),
            pl.BlockSpec((TR, KE), lambda i: (i, 0)),
            pl.BlockSpec((KE, K), lambda i: (0, 0)),
            pl.BlockSpec((1, K), lambda i: (0, 0)),
            pl.BlockSpec((J, K, 128), lambda i: (0, 0, 0)),
            pl.BlockSpec((IG, 128), lambda i: (0, 0)),
        ],
        out_specs=pl.BlockSpec((TR, D), lambda i: (i, 0)),
        out_shape=jax.ShapeDtypeStruct((ep, D), jnp.float32),
    )(G, ea, lw, lb, Wg, Sg)


def _agg_gru(s0, s1, d0, d1, cb, h, WihT, WhhT, bih, bhh, D, TR):
    n = h.shape[0]

    def body(s0_ref, s1_ref, d0_ref, d1_ref, cb_ref, h_ref,
             wi_ref, wh_ref, bi_ref, bh_ref, o_ref):
        deg = jnp.maximum(d0_ref[:, 0:1] + d1_ref[:, 0:1], 1.0)
        m = _rrelu((s0_ref[...] + s1_ref[...]) / deg + cb_ref[...])
        hv = h_ref[...]
        gi = _bdot(m, wi_ref[...]) + bi_ref[...]
        gh = _bdot(hv, wh_ref[...]) + bh_ref[...]
        r = jax.nn.sigmoid(gi[:, :D] + gh[:, :D])
        z = jax.nn.sigmoid(gi[:, D:2 * D] + gh[:, D:2 * D])
        nn = jnp.tanh(gi[:, 2 * D:] + r * gh[:, 2 * D:])
        o_ref[...] = (1.0 - z) * nn + z * hv

    return pl.pallas_call(
        body,
        grid=(n // TR,),
        in_specs=[
            pl.BlockSpec((TR, D), lambda i: (i, 0)),
            pl.BlockSpec((TR, D), lambda i: (i, 0)),
            pl.BlockSpec((TR, 16), lambda i: (i, 0)),
            pl.BlockSpec((TR, 16), lambda i: (i, 0)),
            pl.BlockSpec((1, D), lambda i: (0, 0)),
            pl.BlockSpec((TR, D), lambda i: (i, 0)),
            pl.BlockSpec((D, 3 * D), lambda i: (0, 0)),
            pl.BlockSpec((D, 3 * D), lambda i: (0, 0)),
            pl.BlockSpec((1, 3 * D), lambda i: (0, 0)),
            pl.BlockSpec((1, 3 * D), lambda i: (0, 0)),
        ],
        out_specs=pl.BlockSpec((TR, D), lambda i: (i, 0)),
        out_shape=jax.ShapeDtypeStruct((n, D), jnp.float32),
    )(s0, s1, d0, d1, cb, h, WihT, WhhT, bih, bhh)


def _mid(h, stats, g, b, W1T, b1, W2T, b2, TR):
    n, d = h.shape
    do = W1T.shape[1]

    def body(h_ref, st_ref, g_ref, b_ref, w1_ref, b1_ref, w2_ref, b2_ref, o_ref):
        mean, inv = _bn_from_stats(st_ref, float(n))
        xn = (h_ref[...] - mean) * inv * g_ref[...] + b_ref[...]
        y = _rrelu(_bdot(xn, w1_ref[...]) + b1_ref[...])
        o_ref[...] = _rrelu(_bdot(y, w2_ref[...]) + b2_ref[...])

    return pl.pallas_call(
        body,
        grid=(n // TR,),
        in_specs=[
            pl.BlockSpec((TR, d), lambda i: (i, 0)),
            pl.BlockSpec((8, d), lambda i: (0, 0)),
            pl.BlockSpec((1, d), lambda i: (0, 0)),
            pl.BlockSpec((1, d), lambda i: (0, 0)),
            pl.BlockSpec((d, do), lambda i: (0, 0)),
            pl.BlockSpec((1, do), lambda i: (0, 0)),
            pl.BlockSpec((do, do), lambda i: (0, 0)),
            pl.BlockSpec((1, do), lambda i: (0, 0)),
        ],
        out_specs=pl.BlockSpec((TR, do), lambda i: (i, 0)),
        out_shape=jax.ShapeDtypeStruct((n, do), jnp.float32),
    )(h, stats, g, b, W1T, b1, W2T, b2)


def _set2set(out, batch2d, WihT, WhhT, bih, bhh, TR):
    n, d = out.shape

    def body(o_ref, b_ref, wi_ref, wh_ref, bi_ref, bh_ref, qs_out,
             hx_s, cx_s, qs_s, emax_s, den_s, r_s, e_s):
        s = pl.program_id(0)
        p = pl.program_id(1)
        t = pl.program_id(2)
        nt = pl.num_programs(2)

        @pl.when((s == 0) & (p == 0) & (t == 0))
        def _():
            hx_s[...] = jnp.zeros_like(hx_s)
            cx_s[...] = jnp.zeros_like(cx_s)
            qs_s[...] = jnp.zeros_like(qs_s)

        @pl.when((p == 0) & (t == 0))
        def _():
            g = (_bdot(qs_s[...], wi_ref[...]) + bi_ref[...]
                 + _bdot(hx_s[...], wh_ref[...]) + bh_ref[...])
            i = jax.nn.sigmoid(g[:, :d])
            f = jax.nn.sigmoid(g[:, d:2 * d])
            gg = jnp.tanh(g[:, 2 * d:3 * d])
            o = jax.nn.sigmoid(g[:, 3 * d:])
            c2 = f * cx_s[...] + i * gg
            cx_s[...] = c2
            hx_s[...] = o * jnp.tanh(c2)
            emax_s[...] = jnp.full_like(emax_s, -1e30)
            den_s[...] = jnp.zeros_like(den_s)
            r_s[...] = jnp.zeros_like(r_s)

        ov = o_ref[...]
        oh = (b_ref[...] == lax.broadcasted_iota(jnp.int32, (TR, NG), 1)
              ).astype(jnp.float32)

        @pl.when(p == 0)
        def _():
            qb = jnp.dot(oh, hx_s[...], preferred_element_type=jnp.float32, precision=lax.Precision.HIGHEST)
            e0 = jnp.sum(ov * qb, axis=1, keepdims=True)
            e_s[pl.ds(t * TR, TR), :] = e0
            em = jnp.max(jnp.where(oh > 0, e0, -1e30), axis=0, keepdims=True)
            emax_s[0:1, :] = jnp.maximum(emax_s[0:1, :], em)

        e = e_s[pl.ds(t * TR, TR), :]

        @pl.when(p == 1)
        def _():
            emb = jnp.dot(oh, emax_s[0:1, :].reshape(NG, 1),
                          preferred_element_type=jnp.float32, precision=lax.Precision.HIGHEST)
            ex = jnp.exp(e - emb)
            dpart = lax.dot_general(ex, oh, (((0,), (0,)), ((), ())),
                                    preferred_element_type=jnp.float32, precision=lax.Precision.HIGHEST)
            den_s[0:1, :] += dpart

        @pl.when(p == 2)
        def _():
            emb = jnp.dot(oh, emax_s[0:1, :].reshape(NG, 1),
                          preferred_element_type=jnp.float32, precision=lax.Precision.HIGHEST)
            ex = jnp.exp(e - emb)
            dnb = jnp.dot(oh, den_s[0:1, :].reshape(NG, 1),
                          preferred_element_type=jnp.float32, precision=lax.Precision.HIGHEST)
            a = ex / (dnb + 1e-16)
            rpart = lax.dot_general(oh, a * ov, (((0,), (0,)), ((), ())),
                                    preferred_element_type=jnp.float32, precision=lax.Precision.HIGHEST)
            r_s[...] += rpart

            @pl.when(t == nt - 1)
            def _():
                qs_s[:, :d] = hx_s[...]
                qs_s[:, d:] = r_s[...]
                qs_out[...] = qs_s[...]

    return pl.pallas_call(
        body,
        grid=(3, 3, n // TR),
        in_specs=[
            pl.BlockSpec((TR, d), lambda s, p, t: (t, 0)),
            pl.BlockSpec((TR, 1), lambda s, p, t: (t, 0)),
            pl.BlockSpec((2 * d, 4 * d), lambda s, p, t: (0, 0)),
            pl.BlockSpec((d, 4 * d), lambda s, p, t: (0, 0)),
            pl.BlockSpec((1, 4 * d), lambda s, p, t: (0, 0)),
            pl.BlockSpec((1, 4 * d), lambda s, p, t: (0, 0)),
        ],
        out_specs=pl.BlockSpec((NG, 2 * d), lambda s, p, t: (0, 0)),
        out_shape=jax.ShapeDtypeStruct((NG, 2 * d), jnp.float32),
        scratch_shapes=[
            pltpu.VMEM((NG, d), jnp.float32),
            pltpu.VMEM((NG, d), jnp.float32),
            pltpu.VMEM((NG, 2 * d), jnp.float32),
            pltpu.VMEM((8, NG), jnp.float32),
            pltpu.VMEM((8, NG), jnp.float32),
            pltpu.VMEM((NG, d), jnp.float32),
            pltpu.VMEM((n, 1), jnp.float32),
        ],
    )(out, batch2d, WihT, WhhT, bih, bhh)


def _expand_cat(out, batch2d, qs, TR):
    n, d = out.shape

    def body(o_ref, b_ref, qs_ref, cat_ref):
        oh = (b_ref[...] == lax.broadcasted_iota(jnp.int32, (TR, NG), 1)
              ).astype(jnp.float32)
        pooln = jnp.dot(oh, qs_ref[...], preferred_element_type=jnp.float32, precision=lax.Precision.HIGHEST)
        cat_ref[:, :d] = o_ref[...]
        cat_ref[:, d:] = pooln

    return pl.pallas_call(
        body,
        grid=(n // TR,),
        in_specs=[
            pl.BlockSpec((TR, d), lambda i: (i, 0)),
            pl.BlockSpec((TR, 1), lambda i: (i, 0)),
            pl.BlockSpec((NG, 2 * d), lambda i: (0, 0)),
        ],
        out_specs=pl.BlockSpec((TR, 3 * d), lambda i: (i, 0)),
        out_shape=jax.ShapeDtypeStruct((n, 3 * d), jnp.float32),
    )(out, batch2d, qs)


def _yhat_tile(t0cat, t1):
    d = 64
    t0 = t0cat[:, :d]
    p3 = t0cat[:, d:]
    return jnp.concatenate(
        [(t0 + t1) * 0.5, t0 * t1, (t0 - t1) ** 2, p3], axis=1)


def _final_stats(T0cat, T1, TR):
    n = T0cat.shape[0]

    def body(t0_ref, t1_ref, o_ref):
        @pl.when(pl.program_id(0) == 0)
        def _():
            o_ref[...] = jnp.zeros_like(o_ref)
        y = _yhat_tile(t0_ref[...], t1_ref[...])
        o_ref[0:1, :] += jnp.sum(y, axis=0, keepdims=True)
        o_ref[1:2, :] += jnp.sum(y * y, axis=0, keepdims=True)

    return pl.pallas_call(
        body,
        grid=(n // TR,),
        in_specs=[
            pl.BlockSpec((TR, 192), lambda i: (i, 0)),
            pl.BlockSpec((TR, 64), lambda i: (i, 0)),
        ],
        out_specs=pl.BlockSpec((8, 320), lambda i: (0, 0)),
        out_shape=jax.ShapeDtypeStruct((8, 320), jnp.float32),
    )(T0cat, T1)


def _final_head(T0cat, T1, stats, ng, nb, ea3, WwT, WbT, TR):
    n = T0cat.shape[0]

    def body(t0_ref, t1_ref, st_ref, g_ref, b_ref, ea_ref, ww_ref, wb_ref, o_ref):
        y = _yhat_tile(t0_ref[...], t1_ref[...])
        mean, inv = _bn_from_stats(st_ref, float(n))
        y = (y - mean) * inv * g_ref[...] + b_ref[...]
        eav = ea_ref[...]
        w = _bdot(eav, ww_ref[...])
        bb = _bdot(eav, wb_ref[...])
        o_ref[...] = jnp.sum(y * w, axis=1, keepdims=True) + bb

    return pl.pallas_call(
        body,
        grid=(n // TR,),
        in_specs=[
            pl.BlockSpec((TR, 192), lambda i: (i, 0)),
            pl.BlockSpec((TR, 64), lambda i: (i, 0)),
            pl.BlockSpec((8, 320), lambda i: (0, 0)),
            pl.BlockSpec((1, 320), lambda i: (0, 0)),
            pl.BlockSpec((1, 320), lambda i: (0, 0)),
            pl.BlockSpec((TR, 8), lambda i: (i, 0)),
            pl.BlockSpec((8, 320), lambda i: (0, 0)),
            pl.BlockSpec((8, 1), lambda i: (0, 0)),
        ],
        out_specs=pl.BlockSpec((TR, 1), lambda i: (i, 0)),
        out_shape=jax.ShapeDtypeStruct((n, 1), jnp.float32),
    )(T0cat, T1, stats, ng, nb, ea3, WwT, WbT)


def _pad_idx(a, ep, fill):
    return jnp.pad(a, (0, ep - a.shape[0]), constant_values=fill)


def kernel(x, edge_attr, edge_attr3, params, edge_index, edge_index3, batch):
    p = params
    f32 = jnp.float32

    src = _pad_idx(edge_index[0].astype(jnp.int32), E1P, 0).reshape(-1, CHUNK)
    dst = _pad_idx(edge_index[1].astype(jnp.int32), E1P, N).reshape(-1, CHUNK)
    src3 = edge_index3[0].astype(jnp.int32)
    dst3 = edge_index3[1].astype(jnp.int32)
    src3d = _pad_idx(jnp.concatenate([src3, dst3]), E3DP, 0).reshape(-1, CHUNK)
    dst3d = _pad_idx(jnp.concatenate([dst3, src3]), E3DP, N).reshape(-1, CHUNK)
    src3p = _pad_idx(src3, E3P, 0).reshape(-1, CHUNK)
    dst3p = _pad_idx(dst3, E3P, 0).reshape(-1, CHUNK)

    Wt1 = p['nn1_W']
    Wt2 = p['nn2_W']

    def row(v):
        return v.reshape(1, -1).astype(f32)

    ea3d = jnp.concatenate([edge_attr3, edge_attr3], axis=0)

    xst = _colstats(x, 1000)
    out = _pre_node(x, xst, row(p['norm_x_g']), row(p['norm_x_b']),
                    p['lin_node_W'].T, row(p['lin_node_b']), 1000)

    degp = _sc_degree(dst, E1P)
    d0 = degp[0, :N]
    d1 = degp[1, :N]
    h = out
    for _ in range(2):
        G = _sc_gather(out, src, E1P, DIM)
        msg = _msg(G, edge_attr, Wt1, 12, DIM, 512,
                   lin=(p['lin_edge_W'].T, row(p['lin_edge_b'])))
        sp = _sc_scatter_add(msg, dst, E1P, DIM, False)
        h = _agg_gru(sp[0, :N], sp[1, :N], d0, d1, row(p['conv1_b']), h,
                     p['gru1_Wih'].T, p['gru1_Whh'].T,
                     row(p['gru1_bih']), row(p['gru1_bhh']), DIM, 1000)
        out = h

    hst = _colstats(out, 1000)
    out = _mid(out, hst, row(p['cov_bn_g']), row(p['cov_bn_b']),
               p['cov_l1_W'].T, row(p['cov_l1_b']),
               p['cov_l2_W'].T, row(p['cov_l2_b']), 1000)

    degp3 = _sc_degree(dst3d, E3DP)
    d30 = degp3[0, :N]
    d31 = degp3[1, :N]
    h = out
    for _ in range(2):
        G = _sc_gather(out, src3d, E3DP, 2 * DIM)
        msg = _msg(G, ea3d, Wt2, 8, 2 * DIM, 512)
        sp = _sc_scatter_add(msg, dst3d, E3DP, 2 * DIM, False)
        h = _agg_gru(sp[0, :N], sp[1, :N], d30, d31, row(p['conv2_b']), h,
                     p['gru2_Wih'].T, p['gru2_Whh'].T,
                     row(p['gru2_bih']), row(p['gru2_bhh']), 2 * DIM, 1000)
        out = h

    batch2d = batch.astype(jnp.int32).reshape(N, 1)
    qs = _set2set(out, batch2d, p['s2s_Wih'].T, p['s2s_Whh'].T,
                  row(p['s2s_bih']), row(p['s2s_bhh']), 2000)
    cat = _expand_cat(out, batch2d, qs, 1000)

    T0cat = _sc_gather(cat, src3p, E3P, 192)[:N]
    T1 = _sc_gather(out, dst3p, E3P, 2 * DIM)[:N]

    fst = _final_stats(T0cat, T1, 1000)
    y = _final_head(T0cat, T1, fst, row(p['norm_g']), row(p['norm_b']),
                    edge_attr3, p['lin_weight_W'].T, p['lin_bias_W'].T, 1000)
    return y[:, 0]
